# baseline (speedup 1.0000x reference)
import functools

import jax
import jax.numpy as jnp
from jax import lax
from jax.experimental import pallas as pl
from jax.experimental.pallas import tpu as pltpu
from jax.experimental.pallas import tpu_sc as plsc

D_MODEL = 768
D_FF = 3072
E = 8
K = 2
T = 256

NC = 2
NS = 16
NW = NC * NS


def _router_kernel(x_ref, wr_ref, gd_ref, dst_ref, nsel_ref):
    x = x_ref[...]
    logits = jnp.dot(x, wr_ref[...], preferred_element_type=jnp.float32)
    n = logits.shape[0]
    iota_e = jax.lax.broadcasted_iota(jnp.int32, (1, E), 1)
    big = jnp.int32(E)

    m1 = jnp.max(logits, axis=1, keepdims=True)
    eq1 = logits == m1
    i1 = jnp.min(jnp.where(eq1, iota_e, big), axis=1, keepdims=True)
    sel1 = (iota_e == i1)
    logits2 = jnp.where(sel1, -jnp.inf, logits)
    m2 = jnp.max(logits2, axis=1, keepdims=True)
    eq2 = logits2 == m2
    i2 = jnp.min(jnp.where(eq2, iota_e, big), axis=1, keepdims=True)
    sel2 = (iota_e == i2)

    z = jnp.exp(m2 - m1)
    p1 = 1.0 / (1.0 + z)
    p2 = z / (1.0 + z)
    gd_ref[...] = jnp.where(sel1, p1, 0.0) + jnp.where(sel2, p2, 0.0)

    sel = (sel1 | sel2).astype(jnp.float32)

    c = 256
    iota_r = jax.lax.broadcasted_iota(jnp.int32, (c, c), 0)
    iota_c = jax.lax.broadcasted_iota(jnp.int32, (c, c), 1)
    ltri = (iota_r > iota_c).astype(jnp.float32)
    running = jnp.zeros((1, E), jnp.float32)
    chunks = []
    for ci in range(n // c):
        s = jax.lax.slice(sel, (ci * c, 0), (ci * c + c, E))
        ex = jnp.dot(ltri, s, preferred_element_type=jnp.float32) + running
        chunks.append(ex)
        running = running + jnp.sum(s, axis=0, keepdims=True)
    ex_all = jnp.concatenate(chunks, axis=0)
    nsel_ref[...] = running.astype(jnp.int32)

    r1 = jnp.sum(jnp.where(sel1, ex_all, 0.0), axis=1, keepdims=True)
    r2 = jnp.sum(jnp.where(sel2, ex_all, 0.0), axis=1, keepdims=True)
    d1 = i1 * n + r1.astype(jnp.int32)
    d2 = i2 * n + r2.astype(jnp.int32)
    dst_ref[...] = jnp.concatenate([d1, d2], axis=1).T


DC = 4


def _dispatch_kernel(x_hbm, dst_hbm, xg_hbm, dst_v, rows_v, gsem, ssem):
    n = x_hbm.shape[0]
    cb = dst_v.shape[1]
    pw = DC * cb
    wid = lax.axis_index("s") * NC + lax.axis_index("c")
    base = wid * pw
    bmod = lax.rem(base, n)
    pltpu.sync_copy(dst_hbm.at[wid], dst_v)

    def _gather(c):
        return pltpu.make_async_copy(
            x_hbm.at[pl.ds(bmod + c * cb, cb)],
            rows_v.at[pl.ds(c * cb, cb)], gsem)

    def _scatter(c):
        return pltpu.make_async_copy(
            rows_v.at[pl.ds(c * cb, cb)], xg_hbm.at[dst_v.at[c]], ssem)

    for c in range(DC):
        _gather(c).start()
    for c in range(DC):
        _gather(c).wait()
        _scatter(c).start()
    for c in range(DC):
        _scatter(c).wait()


def _ffn_kernel(nsel_ref, xg_ref, gd_ref, w1_hbm, b1_ref, w2_hbm,
                b2_ref, g_ref, b_ref, out_ref, w1buf, w2buf, sems):
    e = pl.program_id(0)
    rb = pl.program_id(1)

    hf = D_FF // 2

    def _copies(ei, slot):
        return [
            pltpu.make_async_copy(w1_hbm.at[ei, :, pl.ds(0, hf)],
                                  w1buf.at[slot, :, pl.ds(0, hf)],
                                  sems.at[slot, 0]),
            pltpu.make_async_copy(w1_hbm.at[ei, :, pl.ds(hf, hf)],
                                  w1buf.at[slot, :, pl.ds(hf, hf)],
                                  sems.at[slot, 1]),
            pltpu.make_async_copy(w2_hbm.at[ei, pl.ds(0, hf), :],
                                  w2buf.at[slot, pl.ds(0, hf), :],
                                  sems.at[slot, 2]),
            pltpu.make_async_copy(w2_hbm.at[ei, pl.ds(hf, hf), :],
                                  w2buf.at[slot, pl.ds(hf, hf), :],
                                  sems.at[slot, 3]),
        ]

    def _copy(ei, slot):
        for c in _copies(ei, slot):
            c.start()

    def _wait(ei, slot):
        for c in _copies(ei, slot):
            c.wait()

    @pl.when(jnp.logical_and(e == 0, rb == 0))
    def _init():
        out_ref[...] = jnp.zeros_like(out_ref)
        _copy(0, 0)
        _copy(1, 1)

    @pl.when(jnp.logical_and(jnp.logical_and(e > 0, e < E - 1), rb == 0))
    def _prefetch():
        _copy(e + 1, (e + 1) % 2)

    nsel_e = nsel_ref[0, e]
    r0 = rb * T

    @pl.when(jnp.logical_and(rb == 0, nsel_e == 0))
    def _drain_unused():
        _wait(e, e % 2)

    @pl.when(r0 < nsel_e)
    def _active():
        cs = _copies(e, e % 2)

        @pl.when(rb == 0)
        def _await_w1():
            cs[0].wait()
            cs[1].wait()

        rows = r0 + jax.lax.broadcasted_iota(jnp.int32, (T, 1), 0)
        xg = xg_ref[0]
        h1 = jnp.dot(xg, w1buf[e % 2], preferred_element_type=jnp.float32)
        h1 = jnp.maximum(h1 + b1_ref[0], 0.0)

        @pl.when(rb == 0)
        def _await_w2():
            cs[2].wait()
            cs[3].wait()

        h2 = jnp.dot(h1, w2buf[e % 2], preferred_element_type=jnp.float32)
        h = xg + h2 + b2_ref[0]
        mu = jnp.mean(h, axis=-1, keepdims=True)
        var = jnp.mean((h - mu) ** 2, axis=-1, keepdims=True)
        y = (h - mu) / jnp.sqrt(var + 1e-6) * g_ref[0] + b_ref[0]

        iota_e = jax.lax.broadcasted_iota(jnp.int32, (T, E), 1)
        gcol = jnp.sum(jnp.where(iota_e == e, gd_ref[...], 0.0), axis=1,
                       keepdims=True)
        contrib = jnp.where(rows < nsel_e, y * gcol, 0.0)
        out_ref[pl.ds(r0, T), :] += contrib


@jax.jit
def kernel(x, Wr, W1, b1, W2, b2, gamma, beta):
    B, S, D = x.shape
    N = B * S
    xf = x.reshape(N, D)

    gd, dst, nsel = pl.pallas_call(
        _router_kernel,
        out_shape=(
            jax.ShapeDtypeStruct((N, E), jnp.float32),
            jax.ShapeDtypeStruct((K, N), jnp.int32),
            jax.ShapeDtypeStruct((1, E), jnp.int32),
        ),
    )(xf, Wr)

    pw = (N * K) // NW
    dst_sc = dst.reshape(NW, DC, pw // DC)
    dispatch = functools.partial(
        pl.kernel,
        out_type=jax.ShapeDtypeStruct((E * N, D), jnp.float32),
        mesh=plsc.VectorSubcoreMesh(core_axis_name="c", subcore_axis_name="s"),
        scratch_types=[
            pltpu.VMEM((DC, pw // DC), jnp.int32),
            pltpu.VMEM((pw, D), jnp.float32),
            pltpu.SemaphoreType.DMA,
            pltpu.SemaphoreType.DMA,
        ],
    )(_dispatch_kernel)
    xg = dispatch(xf, dst_sc).reshape(E, N, D)

    out = pl.pallas_call(
        _ffn_kernel,
        grid=(E, N // T),
        in_specs=[
            pl.BlockSpec(memory_space=pltpu.SMEM),
            pl.BlockSpec((1, T, D), lambda e, rb: (e, rb, 0)),
            pl.BlockSpec((T, E), lambda e, rb: (rb, 0)),
            pl.BlockSpec(memory_space=pl.ANY),
            pl.BlockSpec((1, 1, D_FF), lambda e, rb: (e, 0, 0)),
            pl.BlockSpec(memory_space=pl.ANY),
            pl.BlockSpec((1, 1, D), lambda e, rb: (e, 0, 0)),
            pl.BlockSpec((1, 1, D), lambda e, rb: (e, 0, 0)),
            pl.BlockSpec((1, 1, D), lambda e, rb: (e, 0, 0)),
        ],
        out_specs=pl.BlockSpec((N, D), lambda e, rb: (0, 0)),
        out_shape=jax.ShapeDtypeStruct((N, D), jnp.float32),
        scratch_shapes=[
            pltpu.VMEM((2, D, D_FF), jnp.float32),
            pltpu.VMEM((2, D_FF, D), jnp.float32),
            pltpu.SemaphoreType.DMA((2, 4)),
        ],
    )(nsel, xg, gd, W1, b1.reshape(E, 1, D_FF), W2,
      b2.reshape(E, 1, D), gamma.reshape(E, 1, D), beta.reshape(E, 1, D))

    return out.reshape(B, S, D)


# speedup vs baseline: 1.1416x; 1.1416x over previous
import functools

import jax
import jax.numpy as jnp
from jax import lax
from jax.experimental import pallas as pl
from jax.experimental.pallas import tpu as pltpu
from jax.experimental.pallas import tpu_sc as plsc

D_MODEL = 768
D_FF = 3072
E = 8
K = 2
T = 512

NC = 2
NS = 16
NW = NC * NS


def _router_kernel(x_ref, wr_ref, gd_ref, dst_ref, nsel_ref):
    x = x_ref[...]
    logits = jnp.dot(x, wr_ref[...], preferred_element_type=jnp.float32)
    n = logits.shape[0]
    iota_e = jax.lax.broadcasted_iota(jnp.int32, (1, E), 1)
    big = jnp.int32(E)

    m1 = jnp.max(logits, axis=1, keepdims=True)
    eq1 = logits == m1
    i1 = jnp.min(jnp.where(eq1, iota_e, big), axis=1, keepdims=True)
    sel1 = (iota_e == i1)
    logits2 = jnp.where(sel1, -jnp.inf, logits)
    m2 = jnp.max(logits2, axis=1, keepdims=True)
    eq2 = logits2 == m2
    i2 = jnp.min(jnp.where(eq2, iota_e, big), axis=1, keepdims=True)
    sel2 = (iota_e == i2)

    z = jnp.exp(m2 - m1)
    p1 = 1.0 / (1.0 + z)
    p2 = z / (1.0 + z)
    gd_ref[...] = jnp.where(sel1, p1, 0.0) + jnp.where(sel2, p2, 0.0)

    sel = (sel1 | sel2).astype(jnp.float32)

    c = 256
    iota_r = jax.lax.broadcasted_iota(jnp.int32, (c, c), 0)
    iota_c = jax.lax.broadcasted_iota(jnp.int32, (c, c), 1)
    ltri = (iota_r > iota_c).astype(jnp.float32)
    running = jnp.zeros((1, E), jnp.float32)
    chunks = []
    for ci in range(n // c):
        s = jax.lax.slice(sel, (ci * c, 0), (ci * c + c, E))
        ex = jnp.dot(ltri, s, preferred_element_type=jnp.float32) + running
        chunks.append(ex)
        running = running + jnp.sum(s, axis=0, keepdims=True)
    ex_all = jnp.concatenate(chunks, axis=0)
    nsel_ref[...] = running.astype(jnp.int32)

    r1 = jnp.sum(jnp.where(sel1, ex_all, 0.0), axis=1, keepdims=True)
    r2 = jnp.sum(jnp.where(sel2, ex_all, 0.0), axis=1, keepdims=True)
    d1 = i1 * n + r1.astype(jnp.int32)
    d2 = i2 * n + r2.astype(jnp.int32)
    dst_ref[...] = jnp.concatenate([d1, d2], axis=1)


def _dispatch_kernel(x_hbm, dst_hbm, xg_hbm, dst_v, rows_v, sem):
    n = x_hbm.shape[0]
    pw = dst_v.shape[0]
    wid = lax.axis_index("s") * NC + lax.axis_index("c")
    base = wid * pw
    pltpu.sync_copy(dst_hbm.at[pl.ds(base, pw)], dst_v)
    pltpu.sync_copy(x_hbm.at[pl.ds(lax.rem(base, n), pw)], rows_v)
    pltpu.async_copy(rows_v, xg_hbm.at[dst_v], sem).wait()


def _ffn_kernel(nsel_ref, xg_ref, gd_ref, w1_hbm, b1_ref, w2_hbm,
                b2_ref, g_ref, b_ref, out_ref, w1buf, w2buf, sems):
    e = pl.program_id(0)
    rb = pl.program_id(1)

    hf = D_FF // 2

    def _copies(ei, slot):
        return [
            pltpu.make_async_copy(w1_hbm.at[ei, :, pl.ds(0, hf)],
                                  w1buf.at[slot, :, pl.ds(0, hf)],
                                  sems.at[slot, 0]),
            pltpu.make_async_copy(w1_hbm.at[ei, :, pl.ds(hf, hf)],
                                  w1buf.at[slot, :, pl.ds(hf, hf)],
                                  sems.at[slot, 1]),
            pltpu.make_async_copy(w2_hbm.at[ei, pl.ds(0, hf), :],
                                  w2buf.at[slot, pl.ds(0, hf), :],
                                  sems.at[slot, 2]),
            pltpu.make_async_copy(w2_hbm.at[ei, pl.ds(hf, hf), :],
                                  w2buf.at[slot, pl.ds(hf, hf), :],
                                  sems.at[slot, 3]),
        ]

    def _copy(ei, slot):
        for c in _copies(ei, slot):
            c.start()

    def _wait(ei, slot):
        for c in _copies(ei, slot):
            c.wait()

    @pl.when(jnp.logical_and(e == 0, rb == 0))
    def _init():
        out_ref[...] = jnp.zeros_like(out_ref)
        _copy(0, 0)
        _copy(1, 1)

    @pl.when(jnp.logical_and(jnp.logical_and(e > 0, e < E - 1), rb == 0))
    def _prefetch():
        _copy(e + 1, (e + 1) % 2)

    @pl.when(rb == 0)
    def _await_weights():
        _wait(e, e % 2)

    nsel_e = nsel_ref[0, e]
    r0 = rb * T

    @pl.when(r0 < nsel_e)
    def _active():
        rows = r0 + jax.lax.broadcasted_iota(jnp.int32, (T, 1), 0)
        xg = xg_ref[0]
        h1 = jnp.dot(xg, w1buf[e % 2], preferred_element_type=jnp.float32)
        h1 = jnp.maximum(h1 + b1_ref[0], 0.0)
        h2 = jnp.dot(h1, w2buf[e % 2], preferred_element_type=jnp.float32)
        h = xg + h2 + b2_ref[0]
        mu = jnp.mean(h, axis=-1, keepdims=True)
        var = jnp.mean((h - mu) ** 2, axis=-1, keepdims=True)
        y = (h - mu) / jnp.sqrt(var + 1e-6) * g_ref[0] + b_ref[0]

        iota_e = jax.lax.broadcasted_iota(jnp.int32, (T, E), 1)
        gcol = jnp.sum(jnp.where(iota_e == e, gd_ref[...], 0.0), axis=1,
                       keepdims=True)
        contrib = jnp.where(rows < nsel_e, y * gcol, 0.0)
        out_ref[pl.ds(r0, T), :] += contrib


@jax.jit
def kernel(x, Wr, W1, b1, W2, b2, gamma, beta):
    B, S, D = x.shape
    N = B * S
    xf = x.reshape(N, D)

    gd, dst, nsel = pl.pallas_call(
        _router_kernel,
        out_shape=(
            jax.ShapeDtypeStruct((N, E), jnp.float32),
            jax.ShapeDtypeStruct((N, K), jnp.int32),
            jax.ShapeDtypeStruct((1, E), jnp.int32),
        ),
    )(xf, Wr)

    dst_sc = dst.T.reshape(N * K)

    pw = (N * K) // NW
    dispatch = functools.partial(
        pl.kernel,
        out_type=jax.ShapeDtypeStruct((E * N, D), jnp.float32),
        mesh=plsc.VectorSubcoreMesh(core_axis_name="c", subcore_axis_name="s"),
        scratch_types=[
            pltpu.VMEM((pw,), jnp.int32),
            pltpu.VMEM((pw, D), jnp.float32),
            pltpu.SemaphoreType.DMA,
        ],
    )(_dispatch_kernel)
    xg = dispatch(xf, dst_sc).reshape(E, N, D)

    out = pl.pallas_call(
        _ffn_kernel,
        grid=(E, N // T),
        in_specs=[
            pl.BlockSpec(memory_space=pltpu.SMEM),
            pl.BlockSpec((1, T, D), lambda e, rb: (e, rb, 0)),
            pl.BlockSpec((T, E), lambda e, rb: (rb, 0)),
            pl.BlockSpec(memory_space=pl.ANY),
            pl.BlockSpec((1, 1, D_FF), lambda e, rb: (e, 0, 0)),
            pl.BlockSpec(memory_space=pl.ANY),
            pl.BlockSpec((1, 1, D), lambda e, rb: (e, 0, 0)),
            pl.BlockSpec((1, 1, D), lambda e, rb: (e, 0, 0)),
            pl.BlockSpec((1, 1, D), lambda e, rb: (e, 0, 0)),
        ],
        out_specs=pl.BlockSpec((N, D), lambda e, rb: (0, 0)),
        out_shape=jax.ShapeDtypeStruct((N, D), jnp.float32),
        scratch_shapes=[
            pltpu.VMEM((2, D, D_FF), jnp.float32),
            pltpu.VMEM((2, D_FF, D), jnp.float32),
            pltpu.SemaphoreType.DMA((2, 4)),
        ],
    )(nsel, xg, gd, W1, b1.reshape(E, 1, D_FF), W2,
      b2.reshape(E, 1, D), gamma.reshape(E, 1, D), beta.reshape(E, 1, D))

    return out.reshape(B, S, D)


# speedup vs baseline: 1.1420x; 1.0003x over previous
import functools

import jax
import jax.numpy as jnp
from jax import lax
from jax.experimental import pallas as pl
from jax.experimental.pallas import tpu as pltpu
from jax.experimental.pallas import tpu_sc as plsc

D_MODEL = 768
D_FF = 3072
E = 8
K = 2
T = 512

NC = 2
NS = 16
NW = NC * NS


def _router_kernel(x_ref, wr_ref, gd_ref, dst_ref, nsel_ref):
    x = x_ref[...]
    logits = jnp.dot(x, wr_ref[...], preferred_element_type=jnp.float32)
    n = logits.shape[0]
    iota_e = jax.lax.broadcasted_iota(jnp.int32, (1, E), 1)
    big = jnp.int32(E)

    m1 = jnp.max(logits, axis=1, keepdims=True)
    eq1 = logits == m1
    i1 = jnp.min(jnp.where(eq1, iota_e, big), axis=1, keepdims=True)
    sel1 = (iota_e == i1)
    logits2 = jnp.where(sel1, -jnp.inf, logits)
    m2 = jnp.max(logits2, axis=1, keepdims=True)
    eq2 = logits2 == m2
    i2 = jnp.min(jnp.where(eq2, iota_e, big), axis=1, keepdims=True)
    sel2 = (iota_e == i2)

    z = jnp.exp(m2 - m1)
    p1 = 1.0 / (1.0 + z)
    p2 = z / (1.0 + z)
    gd_ref[...] = jnp.where(sel1, p1, 0.0) + jnp.where(sel2, p2, 0.0)

    sel = (sel1 | sel2).astype(jnp.float32)

    c = 256
    iota_r = jax.lax.broadcasted_iota(jnp.int32, (c, c), 0)
    iota_c = jax.lax.broadcasted_iota(jnp.int32, (c, c), 1)
    ltri = (iota_r > iota_c).astype(jnp.float32)
    running = jnp.zeros((1, E), jnp.float32)
    chunks = []
    for ci in range(n // c):
        s = jax.lax.slice(sel, (ci * c, 0), (ci * c + c, E))
        ex = jnp.dot(ltri, s, preferred_element_type=jnp.float32) + running
        chunks.append(ex)
        running = running + jnp.sum(s, axis=0, keepdims=True)
    ex_all = jnp.concatenate(chunks, axis=0)
    nsel_ref[...] = running.astype(jnp.int32)

    r1 = jnp.sum(jnp.where(sel1, ex_all, 0.0), axis=1, keepdims=True)
    r2 = jnp.sum(jnp.where(sel2, ex_all, 0.0), axis=1, keepdims=True)
    d1 = i1 * n + r1.astype(jnp.int32)
    d2 = i2 * n + r2.astype(jnp.int32)
    dst_ref[...] = jnp.concatenate([d1, d2], axis=1).T


DC = 4


def _dispatch_kernel(x_hbm, dst_hbm, xg_hbm, dst_v, rows_v, gsem, ssem):
    n = x_hbm.shape[0]
    cb = dst_v.shape[1]
    pw = DC * cb
    wid = lax.axis_index("s") * NC + lax.axis_index("c")
    base = wid * pw
    bmod = lax.rem(base, n)
    pltpu.sync_copy(dst_hbm.at[wid], dst_v)

    def _gather(c):
        return pltpu.make_async_copy(
            x_hbm.at[pl.ds(bmod + c * cb, cb)],
            rows_v.at[pl.ds(c * cb, cb)], gsem)

    def _scatter(c):
        return pltpu.make_async_copy(
            rows_v.at[pl.ds(c * cb, cb)], xg_hbm.at[dst_v.at[c]], ssem)

    for c in range(DC):
        _gather(c).start()
    for c in range(DC):
        _gather(c).wait()
        _scatter(c).start()
    for c in range(DC):
        _scatter(c).wait()


def _ffn_kernel(nsel_ref, xg_ref, gd_ref, w1_hbm, b1_ref, w2_hbm,
                b2_ref, g_ref, b_ref, out_ref, w1buf, w2buf, sems):
    e = pl.program_id(0)
    rb = pl.program_id(1)

    hf = D_FF // 2

    def _copies(ei, slot):
        return [
            pltpu.make_async_copy(w1_hbm.at[ei, :, pl.ds(0, hf)],
                                  w1buf.at[slot, :, pl.ds(0, hf)],
                                  sems.at[slot, 0]),
            pltpu.make_async_copy(w1_hbm.at[ei, :, pl.ds(hf, hf)],
                                  w1buf.at[slot, :, pl.ds(hf, hf)],
                                  sems.at[slot, 1]),
            pltpu.make_async_copy(w2_hbm.at[ei, pl.ds(0, hf), :],
                                  w2buf.at[slot, pl.ds(0, hf), :],
                                  sems.at[slot, 2]),
            pltpu.make_async_copy(w2_hbm.at[ei, pl.ds(hf, hf), :],
                                  w2buf.at[slot, pl.ds(hf, hf), :],
                                  sems.at[slot, 3]),
        ]

    def _copy(ei, slot):
        for c in _copies(ei, slot):
            c.start()

    def _wait(ei, slot):
        for c in _copies(ei, slot):
            c.wait()

    @pl.when(jnp.logical_and(e == 0, rb == 0))
    def _init():
        out_ref[...] = jnp.zeros_like(out_ref)
        _copy(0, 0)
        _copy(1, 1)

    @pl.when(jnp.logical_and(jnp.logical_and(e > 0, e < E - 1), rb == 0))
    def _prefetch():
        _copy(e + 1, (e + 1) % 2)

    @pl.when(rb == 0)
    def _await_weights():
        _wait(e, e % 2)

    nsel_e = nsel_ref[0, e]
    r0 = rb * T

    @pl.when(r0 < nsel_e)
    def _active():
        rows = r0 + jax.lax.broadcasted_iota(jnp.int32, (T, 1), 0)
        xg = xg_ref[0]
        h1 = jnp.dot(xg, w1buf[e % 2], preferred_element_type=jnp.float32)
        h1 = jnp.maximum(h1 + b1_ref[0], 0.0)
        h2 = jnp.dot(h1, w2buf[e % 2], preferred_element_type=jnp.float32)
        h = xg + h2 + b2_ref[0]
        mu = jnp.mean(h, axis=-1, keepdims=True)
        var = jnp.mean((h - mu) ** 2, axis=-1, keepdims=True)
        y = (h - mu) / jnp.sqrt(var + 1e-6) * g_ref[0] + b_ref[0]

        iota_e = jax.lax.broadcasted_iota(jnp.int32, (T, E), 1)
        gcol = jnp.sum(jnp.where(iota_e == e, gd_ref[...], 0.0), axis=1,
                       keepdims=True)
        contrib = jnp.where(rows < nsel_e, y * gcol, 0.0)
        out_ref[pl.ds(r0, T), :] += contrib


@jax.jit
def kernel(x, Wr, W1, b1, W2, b2, gamma, beta):
    B, S, D = x.shape
    N = B * S
    xf = x.reshape(N, D)

    gd, dst, nsel = pl.pallas_call(
        _router_kernel,
        out_shape=(
            jax.ShapeDtypeStruct((N, E), jnp.float32),
            jax.ShapeDtypeStruct((K, N), jnp.int32),
            jax.ShapeDtypeStruct((1, E), jnp.int32),
        ),
    )(xf, Wr)

    pw = (N * K) // NW
    dst_sc = dst.reshape(NW, DC, pw // DC)
    dispatch = functools.partial(
        pl.kernel,
        out_type=jax.ShapeDtypeStruct((E * N, D), jnp.float32),
        mesh=plsc.VectorSubcoreMesh(core_axis_name="c", subcore_axis_name="s"),
        scratch_types=[
            pltpu.VMEM((DC, pw // DC), jnp.int32),
            pltpu.VMEM((pw, D), jnp.float32),
            pltpu.SemaphoreType.DMA,
            pltpu.SemaphoreType.DMA,
        ],
    )(_dispatch_kernel)
    xg = dispatch(xf, dst_sc).reshape(E, N, D)

    out = pl.pallas_call(
        _ffn_kernel,
        grid=(E, N // T),
        in_specs=[
            pl.BlockSpec(memory_space=pltpu.SMEM),
            pl.BlockSpec((1, T, D), lambda e, rb: (e, rb, 0)),
            pl.BlockSpec((T, E), lambda e, rb: (rb, 0)),
            pl.BlockSpec(memory_space=pl.ANY),
            pl.BlockSpec((1, 1, D_FF), lambda e, rb: (e, 0, 0)),
            pl.BlockSpec(memory_space=pl.ANY),
            pl.BlockSpec((1, 1, D), lambda e, rb: (e, 0, 0)),
            pl.BlockSpec((1, 1, D), lambda e, rb: (e, 0, 0)),
            pl.BlockSpec((1, 1, D), lambda e, rb: (e, 0, 0)),
        ],
        out_specs=pl.BlockSpec((N, D), lambda e, rb: (0, 0)),
        out_shape=jax.ShapeDtypeStruct((N, D), jnp.float32),
        scratch_shapes=[
            pltpu.VMEM((2, D, D_FF), jnp.float32),
            pltpu.VMEM((2, D_FF, D), jnp.float32),
            pltpu.SemaphoreType.DMA((2, 4)),
        ],
    )(nsel, xg, gd, W1, b1.reshape(E, 1, D_FF), W2,
      b2.reshape(E, 1, D), gamma.reshape(E, 1, D), beta.reshape(E, 1, D))

    return out.reshape(B, S, D)
